# Initial kernel scaffold; baseline (speedup 1.0000x reference)
import jax
import jax.numpy as jnp
from jax.experimental import pallas as pl


def kernel(x_expert, expert_indices, expert_gate, num_tokens, W, b):
    raise NotImplementedError("write your pallas kernel here")


# speedup vs baseline: 1.7313x; 1.7313x over previous
import jax
import jax.numpy as jnp
from jax.experimental import pallas as pl
from jax.experimental.pallas import tpu as pltpu


def _body(x_ref, w_ref, g_ref, ti_ref, bias_ref, out_ref, z_ref):
    E = x_ref.shape[1]
    C = x_ref.shape[2]
    R = g_ref.shape[2]
    TBLK = out_ref.shape[1]

    tb = pl.program_id(1)

    @pl.when(tb == 0)
    def _compute_z():
        bias = bias_ref[...]
        for e in range(E):
            xe = x_ref[0, e]
            we = w_ref[e]
            ze = jnp.dot(xe, we, preferred_element_type=jnp.float32)
            gg = g_ref[0, 0, e * C:(e + 1) * C]
            ze = (ze + bias) * gg[:, None]
            z_ref[e * C:(e + 1) * C, :] = ze.astype(jnp.bfloat16)

    ti = ti_ref[0, 0, :]
    t0 = tb * TBLK
    rows = jax.lax.broadcasted_iota(jnp.int32, (TBLK, R), 0) + t0
    sT = jnp.where(ti[None, :] == rows, 1.0, 0.0).astype(jnp.bfloat16)
    out_ref[0] = jnp.dot(sT, z_ref[...], preferred_element_type=jnp.float32)


def kernel(x_expert, expert_indices, expert_gate, num_tokens, W, b):
    B, E, C, I = x_expert.shape
    O = W.shape[0]
    R = E * C
    T = num_tokens if isinstance(num_tokens, int) else R

    Wt = W.reshape(E, O, I).transpose(0, 2, 1)
    g_f = expert_gate.reshape(B, 1, R)
    ti_f = expert_indices.reshape(B, 1, R)
    bias2 = b.reshape(1, O)

    TBLK = 512
    grid = (B, T // TBLK)

    out = pl.pallas_call(
        _body,
        grid=grid,
        in_specs=[
            pl.BlockSpec((1, E, C, I), lambda bi, ti: (bi, 0, 0, 0)),
            pl.BlockSpec((E, I, O), lambda bi, ti: (0, 0, 0)),
            pl.BlockSpec((1, 1, R), lambda bi, ti: (bi, 0, 0)),
            pl.BlockSpec((1, 1, R), lambda bi, ti: (bi, 0, 0)),
            pl.BlockSpec((1, O), lambda bi, ti: (0, 0)),
        ],
        out_specs=pl.BlockSpec((1, TBLK, O), lambda bi, ti: (bi, ti, 0)),
        out_shape=jax.ShapeDtypeStruct((B, T, O), jnp.float32),
        scratch_shapes=[pltpu.VMEM((R, O), jnp.bfloat16)],
        compiler_params=pltpu.CompilerParams(
            dimension_semantics=("arbitrary", "arbitrary"),
        ),
    )(x_expert, Wt, g_f, ti_f, bias2)
    return out
